# Initial kernel scaffold; baseline (speedup 1.0000x reference)
import jax
import jax.numpy as jnp
from jax.experimental import pallas as pl


def kernel(h, edge_index, vw1, vb1, vw2, vb2, pw1, pb1, pw2, pb2, fw1, fb1, fw2, fb2):
    raise NotImplementedError("write your pallas kernel here")


# speedup vs baseline: 3.1375x; 3.1375x over previous
import functools

import jax
import jax.numpy as jnp
from jax import lax
from jax.experimental import pallas as pl
from jax.experimental.pallas import tpu as pltpu
from jax.experimental.pallas import tpu_sc as plsc

DT = 0.03

_NC = 2
_NS = 16
_NW = _NC * _NS


def _gather_body(h_hbm, row_hbm, col_hbm, hi_hbm, hj_hbm,
                 idx_v, rows_v, sem, *, epw, ch):
    c = lax.axis_index("c")
    s = lax.axis_index("s")
    wid = s * _NC + c
    base = wid * epw

    def body(i, _):
        off = base + i * ch
        pltpu.sync_copy(row_hbm.at[pl.ds(off, ch)], idx_v)
        pltpu.async_copy(h_hbm.at[idx_v], rows_v, sem).wait()
        pltpu.sync_copy(rows_v, hi_hbm.at[pl.ds(off, ch)])
        pltpu.sync_copy(col_hbm.at[pl.ds(off, ch)], idx_v)
        pltpu.async_copy(h_hbm.at[idx_v], rows_v, sem).wait()
        pltpu.sync_copy(rows_v, hj_hbm.at[pl.ds(off, ch)])
        return 0

    lax.fori_loop(0, epw // ch, body, 0)


def _sc_gather(h, row, col):
    e = row.shape[0]
    d = h.shape[1]
    epw = e // _NW
    ch = 400
    mesh = plsc.VectorSubcoreMesh(core_axis_name="c", subcore_axis_name="s")
    kern = pl.kernel(
        functools.partial(_gather_body, epw=epw, ch=ch),
        out_type=(
            jax.ShapeDtypeStruct((e, d), h.dtype),
            jax.ShapeDtypeStruct((e, d), h.dtype),
        ),
        mesh=mesh,
        scratch_types=[
            pltpu.VMEM((ch,), jnp.int32),
            pltpu.VMEM((ch, d), h.dtype),
            pltpu.SemaphoreType.DMA,
        ],
    )
    return kern(h, row, col)


def _scatter_body(msg_hbm, row_hbm, zeros_hbm, out_hbm,
                  idx_v, msg_v, shared, *, epw, ch, nps):
    c = lax.axis_index("c")
    s = lax.axis_index("s")
    wid = s * _NC + c
    base = wid * epw

    pltpu.sync_copy(zeros_hbm, shared.at[pl.ds(s * nps, nps)])
    plsc.subcore_barrier()

    def body(i, _):
        off = base + i * ch
        pltpu.sync_copy(row_hbm.at[pl.ds(off, ch)], idx_v)
        pltpu.sync_copy(msg_hbm.at[pl.ds(off, ch)], msg_v)
        pltpu.sync_copy(msg_v, shared.at[idx_v], add=True)
        return 0

    lax.fori_loop(0, epw // ch, body, 0)
    plsc.subcore_barrier()

    npad = nps * _NS
    pltpu.sync_copy(shared.at[pl.ds(s * nps, nps)],
                    out_hbm.at[pl.ds(c * npad + s * nps, nps)])


def _sc_scatter(msg, row, n_pad):
    e, d = msg.shape
    epw = e // _NW
    ch = 200
    nps = n_pad // _NS
    zeros = jnp.zeros((nps, d), msg.dtype)
    mesh = plsc.VectorSubcoreMesh(core_axis_name="c", subcore_axis_name="s")
    kern = pl.kernel(
        functools.partial(_scatter_body, epw=epw, ch=ch, nps=nps),
        out_type=jax.ShapeDtypeStruct((_NC * n_pad, d), msg.dtype),
        mesh=mesh,
        scratch_types=[
            pltpu.VMEM((ch,), jnp.int32),
            pltpu.VMEM((ch, d), msg.dtype),
            pltpu.VMEM_SHARED((n_pad, d), msg.dtype),
        ],
    )
    return kern(msg, row, zeros)


def _mlp_body(hi_ref, hj_ref, vw1a, vw1b, vb1, vw2r, vb2,
              pw1, pb1, pw2, pb2, fw1a, fw1b, fb1, fw2, fb2, out_ref):
    hi = hi_ref[...]
    hj = hj_ref[...]
    tv = jnp.tanh(jnp.dot(hi, vw1a[...], preferred_element_type=jnp.float32)
                  + jnp.dot(hj, vw1b[...], preferred_element_type=jnp.float32)
                  + vb1[...])
    nu = jnp.sum(tv * vw2r[...], axis=1, keepdims=True) + vb2[...]
    diff = nu * (hj - hi)
    tf = jax.nn.relu(jnp.dot(hi, fw1a[...], preferred_element_type=jnp.float32)
                     + jnp.dot(hj, fw1b[...], preferred_element_type=jnp.float32)
                     + fb1[...])
    force = jnp.dot(tf, fw2[...], preferred_element_type=jnp.float32) + fb2[...]
    tp = jnp.tanh(jnp.dot(hi - hj, pw1[...], preferred_element_type=jnp.float32)
                  + pb1[...])
    pres = jnp.dot(tp, pw2[...], preferred_element_type=jnp.float32) + pb2[...]
    out_ref[...] = diff + force - pres


def _tc_mlp(hi, hj, weights):
    e, d = hi.shape
    be = 2000
    grid = e // be
    row_spec = pl.BlockSpec((be, d), lambda i: (i, 0))
    full = lambda a: pl.BlockSpec(a.shape, lambda i: tuple(0 for _ in a.shape))
    return pl.pallas_call(
        _mlp_body,
        out_shape=jax.ShapeDtypeStruct((e, d), jnp.float32),
        grid=(grid,),
        in_specs=[row_spec, row_spec] + [full(w) for w in weights],
        out_specs=row_spec,
    )(hi, hj, *weights)


def _combine_body(h_ref, p0_ref, p1_ref, out_ref):
    out_ref[...] = h_ref[...] + DT * (p0_ref[...] + p1_ref[...])


def _tc_combine(h, partials, n_pad):
    n, d = h.shape
    bn = 80
    spec = pl.BlockSpec((bn, d), lambda i: (i, 0))
    p1_spec = pl.BlockSpec((bn, d), lambda i: (i + n_pad // bn, 0))
    return pl.pallas_call(
        _combine_body,
        out_shape=jax.ShapeDtypeStruct((n, d), jnp.float32),
        grid=(n // bn,),
        in_specs=[spec, spec, p1_spec],
        out_specs=spec,
    )(h, partials, partials)


def kernel(h, edge_index, vw1, vb1, vw2, vb2, pw1, pb1, pw2, pb2,
           fw1, fb1, fw2, fb2):
    n, d = h.shape
    row = edge_index[0]
    col = edge_index[1]

    hi, hj = _sc_gather(h, row, col)

    weights = (
        vw1[:d], vw1[d:], vb1.reshape(1, d),
        vw2.reshape(1, d), vb2.reshape(1, 1),
        pw1, pb1.reshape(1, d), pw2, pb2.reshape(1, d),
        fw1[:d], fw1[d:], fb1.reshape(1, d), fw2, fb2.reshape(1, d),
    )
    msg = _tc_mlp(hi, hj, weights)

    n_pad = ((n + _NW * 8 - 1) // (_NW * 8)) * (_NW * 8)
    partials = _sc_scatter(msg, row, n_pad)

    return _tc_combine(h, partials, n_pad)
